# baseline (speedup 1.0000x reference)
import functools
from math import sqrt

import jax
import jax.numpy as jnp
from jax import lax
from jax.experimental import pallas as pl
from jax.experimental.pallas import tpu as pltpu
from jax.experimental.pallas import tpu_sc as plsc

VOCAB = 100000
DIM = 128
SCALE = float(sqrt(DIM))

NC = 2
NS = 16
NW = NC * NS

NBATCH = 4096
SEQ = 50

NSLICE = 4
SB = NBATCH // NSLICE
SLICE_ROWS = SB * SEQ
RPT = SLICE_ROWS // NW
GCH = 80
NGC = RPT // GCH

BB = 32

_mesh = plsc.VectorSubcoreMesh(core_axis_name="c", subcore_axis_name="s")


@functools.partial(
    pl.kernel,
    mesh=_mesh,
    out_type=jax.ShapeDtypeStruct((SLICE_ROWS, DIM), jnp.float32),
    scratch_types=[
        pltpu.VMEM((NGC, GCH), jnp.int32),
        pltpu.VMEM((2, GCH, DIM), jnp.float32),
        pltpu.VMEM((2, GCH, DIM), jnp.float32),
        pltpu.SemaphoreType.DMA,
        pltpu.SemaphoreType.DMA,
    ],
)
def _gather_slice(idx_hbm, table_hbm, out_hbm, idx_v, gbuf, obuf, gsem, osem):
    wid = lax.axis_index("s") * NC + lax.axis_index("c")
    base = wid * RPT
    pltpu.sync_copy(idx_hbm.at[wid], idx_v)

    pltpu.async_copy(table_hbm.at[idx_v.at[0]], gbuf.at[0], gsem)
    pltpu.async_copy(table_hbm.at[idx_v.at[1]], gbuf.at[1], gsem)

    def pair_body(p, _):
        c0 = 2 * p
        for b in range(2):
            c = c0 + b
            pltpu.make_async_copy(
                table_hbm.at[idx_v.at[c]], gbuf.at[b], gsem).wait()

            @pl.when(c >= 2)
            def _wait_ocopy():
                pltpu.make_async_copy(
                    obuf.at[b],
                    out_hbm.at[pl.ds(base + (c - 2) * GCH, GCH)],
                    osem).wait()

            def scale_row(i, _):
                for jj in range(DIM // 16):
                    s = pl.ds(jj * 16, 16)
                    obuf[b, i, s] = gbuf[b, i, s] * SCALE
                return 0

            lax.fori_loop(0, GCH, scale_row, 0)

            @pl.when(c + 2 < NGC)
            def _next_gather():
                pltpu.async_copy(
                    table_hbm.at[idx_v.at[c + 2]], gbuf.at[b], gsem)

            pltpu.async_copy(
                obuf.at[b], out_hbm.at[pl.ds(base + c * GCH, GCH)], osem)
        return 0

    lax.fori_loop(0, NGC // 2, pair_body, 0)

    for b in range(2):
        c = NGC - 2 + b
        pltpu.make_async_copy(
            obuf.at[b], out_hbm.at[pl.ds(base + c * GCH, GCH)],
            osem).wait()


def _relayout_body(in_ref, out_ref):
    for bb in range(BB):
        out_ref[bb] = in_ref[pl.ds(bb * SEQ, SEQ), :]


def _relayout_body_prev(in_ref, prev_ref, out_ref):
    del prev_ref
    _relayout_body(in_ref, out_ref)


_OUT_SHAPE = jax.ShapeDtypeStruct((NBATCH, SEQ, DIM), jnp.float32)
_IN_SPEC = pl.BlockSpec((BB * SEQ, DIM), lambda i: (i, 0))


def _assemble(s, part, prev):
    nblk = SB // BB
    out_spec = pl.BlockSpec(
        (BB, SEQ, DIM), lambda i, _s=s: (_s * nblk + i, 0, 0))
    if prev is None:
        return pl.pallas_call(
            _relayout_body,
            grid=(nblk,),
            in_specs=[_IN_SPEC],
            out_specs=out_spec,
            out_shape=_OUT_SHAPE,
        )(part)
    return pl.pallas_call(
        _relayout_body_prev,
        grid=(nblk,),
        in_specs=[_IN_SPEC, pl.BlockSpec(memory_space=pl.ANY)],
        out_specs=out_spec,
        out_shape=_OUT_SHAPE,
        input_output_aliases={1: 0},
    )(part, prev)


def kernel(x, table):
    idx = x.reshape(NSLICE, NW, NGC, GCH).astype(jnp.int32)
    out = None
    for s in range(NSLICE):
        part = _gather_slice(idx[s], table)
        out = _assemble(s, part, out)
    return out
